# baseline (speedup 1.0000x reference)
import jax
import jax.numpy as jnp
from jax.experimental import pallas as pl
from jax.experimental.pallas import tpu as pltpu

_B, _N, _L = 4, 4096, 1024
_R = _B * _N
_CH = 256
_MAXT = 2048
_SCHED = [256, 256, 512, 1024, 2048, 2048, 2048, 2048, 2048, 2048, 1024, 512, 256, 256]
assert sum(_SCHED) == _R


def _scan_tile(in_buf, out_buf, slot, rows, carry):
    rows_i = jax.lax.broadcasted_iota(jnp.int32, (_CH, _CH), 0)
    cols_i = jax.lax.broadcasted_iota(jnp.int32, (_CH, _CH), 1)
    tri = (cols_i < rows_i).astype(jnp.float32)
    for c in range(rows // _CH):
        xc = in_buf[slot, pl.ds(c * _CH, _CH), :]
        excl = jnp.dot(tri, xc, preferred_element_type=jnp.float32)
        out_buf[slot, pl.ds(c * _CH, _CH), :] = excl + carry
        carry = carry + jnp.sum(xc, axis=0, keepdims=True)
    return carry


def _body(x_ref, o_ref, in_buf, out_buf, rsem, wsem):
    T = len(_SCHED)
    starts = [0]
    for r in _SCHED:
        starts.append(starts[-1] + r)

    def rd(t):
        return pltpu.make_async_copy(
            x_ref.at[pl.ds(starts[t], _SCHED[t]), :],
            in_buf.at[t % 2, pl.ds(0, _SCHED[t]), :],
            rsem.at[t % 2],
        )

    def wr(t):
        return pltpu.make_async_copy(
            out_buf.at[t % 2, pl.ds(0, _SCHED[t]), :],
            o_ref.at[pl.ds(starts[t], _SCHED[t]), :],
            wsem.at[t % 2],
        )

    rd(0).start()
    rd(1).start()
    carry = jnp.zeros((1, _L), jnp.float32)
    for t in range(T):
        s = t % 2
        rd(t).wait()
        if t >= 2:
            wr(t - 2).wait()
        if starts[t] % _N == 0:
            carry = jnp.zeros((1, _L), jnp.float32)
        carry = _scan_tile(in_buf, out_buf, s, _SCHED[t], carry)
        wr(t).start()
        if t + 2 < T:
            rd(t + 2).start()
    wr(T - 2).wait()
    wr(T - 1).wait()


def kernel(x):
    x2 = x.reshape(_R, _L)
    out = pl.pallas_call(
        _body,
        in_specs=[pl.BlockSpec(memory_space=pl.ANY)],
        out_specs=pl.BlockSpec(memory_space=pl.ANY),
        out_shape=jax.ShapeDtypeStruct((_R, _L), jnp.float32),
        scratch_shapes=[
            pltpu.VMEM((2, _MAXT, _L), jnp.float32),
            pltpu.VMEM((2, _MAXT, _L), jnp.float32),
            pltpu.SemaphoreType.DMA((2,)),
            pltpu.SemaphoreType.DMA((2,)),
        ],
    )(x2)
    return out.reshape(_B, _N, _L)


# speedup vs baseline: 1.0439x; 1.0439x over previous
import jax
import jax.numpy as jnp
from jax.experimental import pallas as pl
from jax.experimental.pallas import tpu as pltpu

_B, _N, _L = 4, 4096, 1024
_R = _B * _N
_CH = 256
_MAXT = 2048
_SCHED = [512, 512, 1024, 2048, 2048, 2048, 2048, 2048, 2048, 1024, 512, 512]
assert sum(_SCHED) == _R


def _scan_tile(in_buf, out_buf, rslot, wslot, rows, carry):
    rows_i = jax.lax.broadcasted_iota(jnp.int32, (_CH, _CH), 0)
    cols_i = jax.lax.broadcasted_iota(jnp.int32, (_CH, _CH), 1)
    tri = (cols_i < rows_i).astype(jnp.float32)
    for c in range(rows // _CH):
        xc = in_buf[rslot, pl.ds(c * _CH, _CH), :]
        excl = jnp.dot(tri, xc, preferred_element_type=jnp.float32)
        out_buf[wslot, pl.ds(c * _CH, _CH), :] = excl + carry
        carry = carry + jnp.sum(xc, axis=0, keepdims=True)
    return carry


def _body(x_ref, o_ref, in_buf, out_buf, rsem, wsem):
    T = len(_SCHED)
    starts = [0]
    for r in _SCHED:
        starts.append(starts[-1] + r)

    def rd(t):
        return pltpu.make_async_copy(
            x_ref.at[pl.ds(starts[t], _SCHED[t]), :],
            in_buf.at[t % 3, pl.ds(0, _SCHED[t]), :],
            rsem.at[t % 3],
        )

    def wr(t):
        return pltpu.make_async_copy(
            out_buf.at[t % 2, pl.ds(0, _SCHED[t]), :],
            o_ref.at[pl.ds(starts[t], _SCHED[t]), :],
            wsem.at[t % 2],
        )

    rd(0).start()
    rd(1).start()
    rd(2).start()
    carry = jnp.zeros((1, _L), jnp.float32)
    for t in range(T):
        s = t % 2
        rd(t).wait()
        if t >= 2:
            wr(t - 2).wait()
        if starts[t] % _N == 0:
            carry = jnp.zeros((1, _L), jnp.float32)
        carry = _scan_tile(in_buf, out_buf, t % 3, s, _SCHED[t], carry)
        wr(t).start()
        if t + 3 < T:
            rd(t + 3).start()
    wr(T - 2).wait()
    wr(T - 1).wait()


def kernel(x):
    x2 = x.reshape(_R, _L)
    out = pl.pallas_call(
        _body,
        in_specs=[pl.BlockSpec(memory_space=pl.ANY)],
        out_specs=pl.BlockSpec(memory_space=pl.ANY),
        out_shape=jax.ShapeDtypeStruct((_R, _L), jnp.float32),
        scratch_shapes=[
            pltpu.VMEM((3, _MAXT, _L), jnp.float32),
            pltpu.VMEM((2, _MAXT, _L), jnp.float32),
            pltpu.SemaphoreType.DMA((3,)),
            pltpu.SemaphoreType.DMA((2,)),
        ],
    )(x2)
    return out.reshape(_B, _N, _L)


# speedup vs baseline: 1.0604x; 1.0158x over previous
import jax
import jax.numpy as jnp
from jax.experimental import pallas as pl
from jax.experimental.pallas import tpu as pltpu

_B, _N, _L = 4, 4096, 1024
_R = _B * _N
_CH = 256
_MAXT = 2048
_SCHED = [512, 512, 1024, 2048, 2048, 2048, 2048, 2048, 2048, 1024, 512, 512]
assert sum(_SCHED) == _R


def _scan_tile(in_buf, out_buf, rslot, wslot, rows, carry):
    rows_i = jax.lax.broadcasted_iota(jnp.int32, (_CH, _CH), 0)
    cols_i = jax.lax.broadcasted_iota(jnp.int32, (_CH, _CH), 1)
    tri = (cols_i < rows_i).astype(jnp.float32)
    for c in range(rows // _CH):
        xc = in_buf[rslot, pl.ds(c * _CH, _CH), :]
        excl = jnp.dot(tri, xc, preferred_element_type=jnp.float32)
        out_buf[wslot, pl.ds(c * _CH, _CH), :] = excl + carry
        carry = carry + jnp.sum(xc, axis=0, keepdims=True)
    return carry


def _body(x_ref, o_ref, in_buf, out_buf, rsem, wsem):
    T = len(_SCHED)
    starts = [0]
    for r in _SCHED:
        starts.append(starts[-1] + r)

    def rd(t):
        return pltpu.make_async_copy(
            x_ref.at[pl.ds(starts[t], _SCHED[t]), :],
            in_buf.at[t % 4, pl.ds(0, _SCHED[t]), :],
            rsem.at[t % 4],
        )

    def wr(t):
        return pltpu.make_async_copy(
            out_buf.at[t % 3, pl.ds(0, _SCHED[t]), :],
            o_ref.at[pl.ds(starts[t], _SCHED[t]), :],
            wsem.at[t % 3],
        )

    rd(0).start()
    rd(1).start()
    rd(2).start()
    rd(3).start()
    carry = jnp.zeros((1, _L), jnp.float32)
    for t in range(T):
        rd(t).wait()
        if t >= 3:
            wr(t - 3).wait()
        if starts[t] % _N == 0:
            carry = jnp.zeros((1, _L), jnp.float32)
        carry = _scan_tile(in_buf, out_buf, t % 4, t % 3, _SCHED[t], carry)
        wr(t).start()
        if t + 4 < T:
            rd(t + 4).start()
    wr(T - 3).wait()
    wr(T - 2).wait()
    wr(T - 1).wait()


def kernel(x):
    x2 = x.reshape(_R, _L)
    out = pl.pallas_call(
        _body,
        in_specs=[pl.BlockSpec(memory_space=pl.ANY)],
        out_specs=pl.BlockSpec(memory_space=pl.ANY),
        out_shape=jax.ShapeDtypeStruct((_R, _L), jnp.float32),
        scratch_shapes=[
            pltpu.VMEM((4, _MAXT, _L), jnp.float32),
            pltpu.VMEM((3, _MAXT, _L), jnp.float32),
            pltpu.SemaphoreType.DMA((4,)),
            pltpu.SemaphoreType.DMA((3,)),
        ],
    )(x2)
    return out.reshape(_B, _N, _L)
